# baseline (speedup 1.0000x reference)
import jax
import jax.numpy as jnp
from jax import lax
from jax.experimental import pallas as pl
from jax.experimental.pallas import tpu as pltpu

_B, _N, _C = 32, 20000, 81
_NB = 2
_BN = _N // _NB
_ROWS = _B * _NB
_NEG_POS_RATIO = 4.0
_NEG_FOR_HARD = 100.0


def _body(yp_ref, yt_ref, out_ref, z_scr):
    b = pl.program_id(0)
    i = pl.program_id(1)

    yp = yp_ref[0]
    yt = yt_ref[0]
    v = jnp.sum(yt * yp, axis=-1, keepdims=True)
    y0 = yp[:, 0:1]
    bg = yt[:, 0:1]
    z = jnp.where(bg > 0.0, y0, -v)

    r = b * _NB + i
    lane = lax.broadcasted_iota(jnp.int32, (_BN, _ROWS), 1)
    zb = jnp.broadcast_to(z, (_BN, _ROWS))
    z_scr[...] = jnp.where(lane == r, zb, z_scr[...])

    @pl.when((b == _B - 1) & (i == _NB - 1))
    def _():
        za = z_scr[...]
        zi = lax.bitcast_convert_type(za, jnp.int32)
        posm = zi < 0
        posf = posm.astype(jnp.float32)
        cl = -jnp.log(jnp.maximum(jnp.abs(za), 1e-7))

        pos_total = jnp.sum(cl * posf)

        colpos = jnp.sum(posf, axis=0, keepdims=True)
        rowi = lax.broadcasted_iota(jnp.int32, (_ROWS, _ROWS), 0)
        coli = lax.broadcasted_iota(jnp.int32, (_ROWS, _ROWS), 1)
        grp = ((rowi // _NB) == coli).astype(jnp.float32)
        p32 = jnp.dot(colpos, grp,
                      preferred_element_type=jnp.float32)
        lmask = lax.broadcasted_iota(jnp.int32, (1, _ROWS), 1) < _B
        kneg = jnp.sum(jnp.where(
            lmask,
            jnp.minimum(_NEG_POS_RATIO * p32, jnp.float32(_N) - p32),
            0.0))
        denom = jnp.sum(jnp.where(
            lmask, jnp.where(p32 != 0.0, p32, 1.0), 0.0))

        kf = jnp.where(kneg > 0.0, kneg, _NEG_FOR_HARD)
        k = kf.astype(jnp.int32)

        bgm = zi >= 0
        cntbg = jnp.sum(bgm.astype(jnp.int32))

        def tstep(t, T):
            cand = T | jnp.left_shift(jnp.int32(1), 30 - t)
            cnt = jnp.sum((bgm & (zi < cand)).astype(jnp.int32))
            return jnp.where(cnt <= k - 1, cand, T)

        T = lax.fori_loop(0, 31, tstep, jnp.int32(0))

        lt = bgm & (zi < T)
        cnt_lt = jnp.sum(lt.astype(jnp.int32))
        sum_lt = jnp.sum(jnp.where(lt, cl, 0.0))
        rrem = k - cnt_lt

        eq = (zi == T) | ((cntbg < k) & posm)
        fidx = (lax.broadcasted_iota(jnp.int32, (_BN, _ROWS), 1) * _BN
                + lax.broadcasted_iota(jnp.int32, (_BN, _ROWS), 0))

        def istep(t, I):
            cand = I | jnp.left_shift(jnp.int32(1), 20 - t)
            c = jnp.sum((eq & (fidx < cand)).astype(jnp.int32))
            return jnp.where(c <= rrem, cand, I)

        I = lax.fori_loop(0, 21, istep, jnp.int32(0))
        tie_sum = jnp.sum(jnp.where(eq & (fidx < I), cl, 0.0))

        total = (pos_total + sum_lt + tie_sum) / denom
        out_ref[...] = jnp.full((1, 1), total, dtype=jnp.float32)


def _run(y_pred, y_true, interpret=False):
    out = pl.pallas_call(
        _body,
        grid=(_B, _NB),
        in_specs=[
            pl.BlockSpec((1, _BN, _C), lambda b, i: (b, i, 0)),
            pl.BlockSpec((1, _BN, _C), lambda b, i: (b, i, 0)),
        ],
        out_specs=pl.BlockSpec((1, 1), lambda b, i: (0, 0)),
        out_shape=jax.ShapeDtypeStruct((1, 1), jnp.float32),
        scratch_shapes=[
            pltpu.VMEM((_BN, _ROWS), jnp.float32),
        ],
        compiler_params=pltpu.CompilerParams(
            dimension_semantics=("arbitrary", "arbitrary"),
        ),
        interpret=interpret,
    )(y_pred, y_true)
    return out[0, 0]


def kernel(y_pred, y_true):
    return _run(y_pred, y_true)


# speedup vs baseline: 1.0260x; 1.0260x over previous
import jax
import jax.numpy as jnp
from jax import lax
from jax.experimental import pallas as pl
from jax.experimental.pallas import tpu as pltpu

_B, _N, _C = 32, 20000, 81
_NB = 4
_BN = _N // _NB
_ROWS = _B * _NB
_NEG_POS_RATIO = 4.0
_NEG_FOR_HARD = 100.0


def _body(yp_ref, yt_ref, out_ref, z_scr):
    b = pl.program_id(0)
    i = pl.program_id(1)

    yp = yp_ref[0]
    yt = yt_ref[0]
    v = jnp.sum(yt * yp, axis=-1, keepdims=True)
    y0 = yp[:, 0:1]
    bg = yt[:, 0:1]
    z = jnp.where(bg > 0.0, y0, -v)

    r = b * _NB + i
    lane = lax.broadcasted_iota(jnp.int32, (_BN, _ROWS), 1)
    zb = jnp.broadcast_to(z, (_BN, _ROWS))
    z_scr[...] = jnp.where(lane == r, zb, z_scr[...])

    @pl.when((b == _B - 1) & (i == _NB - 1))
    def _():
        za = z_scr[...]
        zi = lax.bitcast_convert_type(za, jnp.int32)
        posm = zi < 0
        posf = posm.astype(jnp.float32)
        cl = -jnp.log(jnp.maximum(jnp.abs(za), 1e-7))

        pos_total = jnp.sum(cl * posf)

        colpos = jnp.sum(posf, axis=0, keepdims=True)
        rowi = lax.broadcasted_iota(jnp.int32, (_ROWS, _ROWS), 0)
        coli = lax.broadcasted_iota(jnp.int32, (_ROWS, _ROWS), 1)
        grp = ((rowi // _NB) == coli).astype(jnp.float32)
        p32 = jnp.dot(colpos, grp,
                      preferred_element_type=jnp.float32)
        lmask = lax.broadcasted_iota(jnp.int32, (1, _ROWS), 1) < _B
        kneg = jnp.sum(jnp.where(
            lmask,
            jnp.minimum(_NEG_POS_RATIO * p32, jnp.float32(_N) - p32),
            0.0))
        denom = jnp.sum(jnp.where(
            lmask, jnp.where(p32 != 0.0, p32, 1.0), 0.0))

        kf = jnp.where(kneg > 0.0, kneg, _NEG_FOR_HARD)
        k = kf.astype(jnp.int32)

        bgm = zi >= 0
        cntbg = jnp.sum(bgm.astype(jnp.int32))

        zc = jnp.where(bgm, zi, jnp.int32(0x7FFFFFFF))

        def tstep(t, T):
            cand = T | jnp.left_shift(jnp.int32(1), 29 - t)
            cnt = jnp.sum((zc < cand).astype(jnp.int32))
            return jnp.where(cnt <= k - 1, cand, T)

        T = lax.fori_loop(0, 30, tstep, jnp.int32(0))
        T = jnp.where(cntbg < k, jnp.int32(0x7FFFFFFF), T)

        lt = zc < T
        cnt_lt = jnp.sum(lt.astype(jnp.int32))
        sum_lt = jnp.sum(jnp.where(lt, cl, 0.0))
        rrem = k - cnt_lt

        eq = (zi == T) | ((cntbg < k) & posm)
        cnt_eq = jnp.sum(eq.astype(jnp.int32))

        def all_ties(_):
            return jnp.sum(jnp.where(eq, cl, 0.0))

        def search_ties(_):
            fidx = (lax.broadcasted_iota(jnp.int32, (_BN, _ROWS), 1) * _BN
                    + lax.broadcasted_iota(jnp.int32, (_BN, _ROWS), 0))

            def istep(t, I):
                cand = I | jnp.left_shift(jnp.int32(1), 20 - t)
                c = jnp.sum((eq & (fidx < cand)).astype(jnp.int32))
                return jnp.where(c <= rrem, cand, I)

            I = lax.fori_loop(0, 21, istep, jnp.int32(0))
            return jnp.sum(jnp.where(eq & (fidx < I), cl, 0.0))

        tie_sum = lax.cond(rrem >= cnt_eq, all_ties, search_ties, 0)

        total = (pos_total + sum_lt + tie_sum) / denom
        out_ref[...] = jnp.full((1, 1), total, dtype=jnp.float32)


def _run(y_pred, y_true, interpret=False):
    out = pl.pallas_call(
        _body,
        grid=(_B, _NB),
        in_specs=[
            pl.BlockSpec((1, _BN, _C), lambda b, i: (b, i, 0)),
            pl.BlockSpec((1, _BN, _C), lambda b, i: (b, i, 0)),
        ],
        out_specs=pl.BlockSpec((1, 1), lambda b, i: (0, 0)),
        out_shape=jax.ShapeDtypeStruct((1, 1), jnp.float32),
        scratch_shapes=[
            pltpu.VMEM((_BN, _ROWS), jnp.float32),
        ],
        compiler_params=pltpu.CompilerParams(
            dimension_semantics=("arbitrary", "arbitrary"),
        ),
        interpret=interpret,
    )(y_pred, y_true)
    return out[0, 0]


def kernel(y_pred, y_true):
    return _run(y_pred, y_true)
